# Initial kernel scaffold; baseline (speedup 1.0000x reference)
import jax
import jax.numpy as jnp
from jax.experimental import pallas as pl


def kernel(features, idx):
    raise NotImplementedError("write your pallas kernel here")


# speedup vs baseline: 1.2707x; 1.2707x over previous
import functools

import jax
import jax.numpy as jnp
from jax import lax
from jax.experimental import pallas as pl
from jax.experimental.pallas import tpu as pltpu
from jax.experimental.pallas import tpu_sc as plsc

B, C, N, M = 8, 64, 50000, 16384
NC, NS, L = 2, 16, 16
NW = NC * NS
WPB = NW // B
CPW = C // WPB
OCHUNK = 4096
NCHUNK = M // OCHUNK


def _gather_body(features, idx, out, idx_v, feat_a, feat_b, out_a, out_b,
                 fsem_a, fsem_b, osem_a, osem_b):
    wid = lax.axis_index("s") * NC + lax.axis_index("c")
    b = wid // WPB
    c0 = (wid % WPB) * CPW

    pltpu.sync_copy(idx.at[pl.ds(b * M, M)], idx_v)

    feat_bufs = (feat_a, feat_b)
    fsems = (fsem_a, fsem_b)
    out_bufs = (out_a, out_b)
    osems = (osem_a, osem_b)

    row0 = (b * C + c0) * N
    pltpu.make_async_copy(
        features.at[pl.ds(row0, N)], feat_a, fsem_a).start()

    pending = [None, None]
    for k in range(CPW):
        fb = feat_bufs[k % 2]
        pltpu.make_async_copy(
            features.at[pl.ds(row0 + k * N, N)], fb, fsems[k % 2]).wait()
        if k + 1 < CPW:
            pltpu.make_async_copy(
                features.at[pl.ds(row0 + (k + 1) * N, N)],
                feat_bufs[(k + 1) % 2], fsems[(k + 1) % 2]).start()

        obase = (b * C + c0 + k) * M
        for h in range(NCHUNK):
            oi = (k * NCHUNK + h) % 2
            ob = out_bufs[oi]
            if pending[oi] is not None:
                pending[oi].wait()

            base = h * OCHUNK

            def body(j, _, fb=fb, ob=ob, base=base):
                iv = idx_v[pl.ds(base + j * L, L)]
                ob[pl.ds(j * L, L)] = plsc.load_gather(fb, [iv])
                return 0

            lax.fori_loop(0, OCHUNK // L, body, 0)

            cp = pltpu.make_async_copy(
                ob, out.at[pl.ds(obase + base, OCHUNK)], osems[oi])
            cp.start()
            pending[oi] = cp

    for cp in pending:
        if cp is not None:
            cp.wait()


@jax.jit
def kernel(features, idx):
    mesh = plsc.VectorSubcoreMesh(core_axis_name="c", subcore_axis_name="s")
    run = functools.partial(
        pl.kernel,
        out_type=jax.ShapeDtypeStruct((B * C * M,), jnp.float32),
        mesh=mesh,
        compiler_params=pltpu.CompilerParams(needs_layout_passes=False),
        scratch_types=[
            pltpu.VMEM((M,), jnp.int32),
            pltpu.VMEM((N,), jnp.float32),
            pltpu.VMEM((N,), jnp.float32),
            pltpu.VMEM((OCHUNK,), jnp.float32),
            pltpu.VMEM((OCHUNK,), jnp.float32),
            pltpu.SemaphoreType.DMA,
            pltpu.SemaphoreType.DMA,
            pltpu.SemaphoreType.DMA,
            pltpu.SemaphoreType.DMA,
        ],
    )(_gather_body)
    out = run(features.reshape(B * C * N), idx.astype(jnp.int32).reshape(B * M))
    return out.reshape(B, C, M)


# speedup vs baseline: 1.6800x; 1.3222x over previous
import functools

import jax
import jax.numpy as jnp
from jax import lax
from jax.experimental import pallas as pl
from jax.experimental.pallas import tpu as pltpu
from jax.experimental.pallas import tpu_sc as plsc

B, C, N, M = 8, 64, 50000, 16384
NC, NS, L = 2, 16, 16
NW = NC * NS
WPB = NW // B
CPW = C // WPB
OCHUNK = 4096
NCHUNK = M // OCHUNK


def _gather_body(features, idx, out, idx_v, feat_a, feat_b, out_a, out_b,
                 fsem_a, fsem_b, osem_a, osem_b):
    wid = lax.axis_index("s") * NC + lax.axis_index("c")
    b = wid // WPB
    c0 = (wid % WPB) * CPW

    pltpu.sync_copy(idx.at[pl.ds(b * M, M)], idx_v)

    feat_bufs = (feat_a, feat_b)
    fsems = (fsem_a, fsem_b)
    out_bufs = (out_a, out_b)
    osems = (osem_a, osem_b)

    row0 = (b * C + c0) * N
    pltpu.make_async_copy(
        features.at[pl.ds(row0, N)], feat_a, fsem_a).start()

    pending = [None, None]
    for k in range(CPW):
        fb = feat_bufs[k % 2]
        pltpu.make_async_copy(
            features.at[pl.ds(row0 + k * N, N)], fb, fsems[k % 2]).wait()
        if k + 1 < CPW:
            pltpu.make_async_copy(
                features.at[pl.ds(row0 + (k + 1) * N, N)],
                feat_bufs[(k + 1) % 2], fsems[(k + 1) % 2]).start()

        obase = (b * C + c0 + k) * M
        for h in range(NCHUNK):
            oi = (k * NCHUNK + h) % 2
            ob = out_bufs[oi]
            if pending[oi] is not None:
                pending[oi].wait()

            base = h * OCHUNK

            @plsc.parallel_loop(0, OCHUNK, step=L, unroll=8)
            def _gather_chunk(i, fb=fb, ob=ob, base=base):
                iv = idx_v[pl.ds(base + i, L)]
                ob[pl.ds(i, L)] = plsc.load_gather(fb, [iv])

            cp = pltpu.make_async_copy(
                ob, out.at[pl.ds(obase + base, OCHUNK)], osems[oi])
            cp.start()
            pending[oi] = cp

    for cp in pending:
        if cp is not None:
            cp.wait()


@jax.jit
def kernel(features, idx):
    mesh = plsc.VectorSubcoreMesh(core_axis_name="c", subcore_axis_name="s")
    run = functools.partial(
        pl.kernel,
        out_type=jax.ShapeDtypeStruct((B * C * M,), jnp.float32),
        mesh=mesh,
        compiler_params=pltpu.CompilerParams(needs_layout_passes=False),
        scratch_types=[
            pltpu.VMEM((M,), jnp.int32),
            pltpu.VMEM((N,), jnp.float32),
            pltpu.VMEM((N,), jnp.float32),
            pltpu.VMEM((OCHUNK,), jnp.float32),
            pltpu.VMEM((OCHUNK,), jnp.float32),
            pltpu.SemaphoreType.DMA,
            pltpu.SemaphoreType.DMA,
            pltpu.SemaphoreType.DMA,
            pltpu.SemaphoreType.DMA,
        ],
    )(_gather_body)
    out = run(features.reshape(B * C * N), idx.astype(jnp.int32).reshape(B * M))
    return out.reshape(B, C, M)


# speedup vs baseline: 5.3879x; 3.2070x over previous
import functools

import jax
import jax.numpy as jnp
from jax import lax
from jax.experimental import pallas as pl
from jax.experimental.pallas import tpu as pltpu
from jax.experimental.pallas import tpu_sc as plsc

B, C, N, M = 8, 64, 50000, 16384
NC, NS, L = 2, 16, 16
NW = NC * NS
WPB = NW // B
CPW = C // WPB
OCHUNK = 4096
NCHUNK = M // OCHUNK


def _gather_body(features, idx, out, idx_v, feat_a, feat_b, out_a, out_b,
                 fsem_a, fsem_b, osem_a, osem_b):
    wid = lax.axis_index("s") * NC + lax.axis_index("c")
    b = wid // WPB
    c0 = (wid % WPB) * CPW

    pltpu.sync_copy(idx.at[b], idx_v)

    feat_bufs = (feat_a, feat_b)
    fsems = (fsem_a, fsem_b)
    out_bufs = (out_a, out_b)
    osems = (osem_a, osem_b)

    pltpu.make_async_copy(features.at[b, c0], feat_a, fsem_a).start()

    pending = [None, None]
    for k in range(CPW):
        fb = feat_bufs[k % 2]
        pltpu.make_async_copy(
            features.at[b, c0 + k], fb, fsems[k % 2]).wait()
        if k + 1 < CPW:
            pltpu.make_async_copy(
                features.at[b, c0 + k + 1],
                feat_bufs[(k + 1) % 2], fsems[(k + 1) % 2]).start()

        for h in range(NCHUNK):
            oi = (k * NCHUNK + h) % 2
            ob = out_bufs[oi]
            if pending[oi] is not None:
                pending[oi].wait()

            base = h * OCHUNK

            @plsc.parallel_loop(0, OCHUNK, step=L, unroll=8)
            def _gather_chunk(i, fb=fb, ob=ob, base=base):
                iv = idx_v[pl.ds(base + i, L)]
                ob[pl.ds(i, L)] = plsc.load_gather(fb, [iv])

            cp = pltpu.make_async_copy(
                ob, out.at[b, c0 + k, pl.ds(base, OCHUNK)], osems[oi])
            cp.start()
            pending[oi] = cp

    for cp in pending:
        if cp is not None:
            cp.wait()


@jax.jit
def kernel(features, idx):
    mesh = plsc.VectorSubcoreMesh(core_axis_name="c", subcore_axis_name="s")
    run = functools.partial(
        pl.kernel,
        out_type=jax.ShapeDtypeStruct((B, C, M), jnp.float32),
        mesh=mesh,
        compiler_params=pltpu.CompilerParams(needs_layout_passes=False),
        scratch_types=[
            pltpu.VMEM((M,), jnp.int32),
            pltpu.VMEM((N,), jnp.float32),
            pltpu.VMEM((N,), jnp.float32),
            pltpu.VMEM((OCHUNK,), jnp.float32),
            pltpu.VMEM((OCHUNK,), jnp.float32),
            pltpu.SemaphoreType.DMA,
            pltpu.SemaphoreType.DMA,
            pltpu.SemaphoreType.DMA,
            pltpu.SemaphoreType.DMA,
        ],
    )(_gather_body)
    return run(features, idx.astype(jnp.int32))
